# baseline (speedup 1.0000x reference)
import functools

import jax
import jax.numpy as jnp
from jax import lax
from jax.experimental import pallas as pl
from jax.experimental.pallas import tpu as pltpu
from jax.experimental.pallas import tpu_sc as plsc

L = 16
NC = 2
NS = 16
NW = NC * NS
ROWS = 50
COLS = 200
COLS_PAD = 208
NCHUNK = COLS_PAD // L
ROWS_PER_W = 2
SLOTS = 8


def _sc_body(hist_hbm, sched_hbm, table_hbm, out_hbm,
             hist_v, idx_v, rows_v, e_v, gsem, hsem):
    cid = lax.axis_index("c")
    sid = lax.axis_index("s")
    wid = sid * NC + cid

    hcopy = pltpu.async_copy(hist_hbm, hist_v, hsem)
    pltpu.sync_copy(sched_hbm.at[wid], idx_v)
    pltpu.async_copy(table_hbm.at[idx_v], rows_v, gsem).wait()
    hcopy.wait()

    lane = lax.broadcasted_iota(jnp.int32, (L,), 0)

    for r in range(ROWS_PER_W):
        row = wid * ROWS_PER_W + r
        rsplat = jnp.full((L,), r, jnp.int32)
        vals = []
        m = jnp.full((L,), -jnp.inf, jnp.float32)
        for c in range(NCHUNK):
            idxc = hist_v[pl.ds(c * L, L)]
            v = plsc.load_gather(rows_v, [rsplat, idxc])
            if (c + 1) * L > COLS:
                v = jnp.where(c * L + lane < COLS, v, -jnp.inf)
            vals.append(v)
            m = jnp.maximum(m, v)
        mmax = jnp.max(m)
        s = jnp.zeros((L,), jnp.float32)
        for c in range(NCHUNK):
            vals[c] = jnp.exp(vals[c] - mmax)
            s = s + vals[c]
        inv = jnp.full((L,), 1.0, jnp.float32) / jnp.broadcast_to(
            jnp.sum(s), (L,))
        for c in range(NCHUNK):
            e_v[pl.ds(c * L, L)] = vals[c] * inv

        @pl.when(row < ROWS)
        def _():
            pltpu.sync_copy(e_v.at[pl.ds(0, COLS)], out_hbm.at[row])


@functools.partial(jax.jit, static_argnums=())
def _run(hist_pad, sched, cid_time):
    mesh = plsc.VectorSubcoreMesh(
        core_axis_name="c", subcore_axis_name="s",
        num_cores=NC, num_subcores=NS)
    fn = pl.kernel(
        _sc_body,
        out_type=jax.ShapeDtypeStruct((ROWS, COLS), jnp.float32),
        mesh=mesh,
        compiler_params=pltpu.CompilerParams(
            needs_layout_passes=False,
            use_tc_tiling_on_sc=False,
        ),
        scratch_types=[
            pltpu.VMEM((COLS_PAD,), jnp.int32),
            pltpu.VMEM((SLOTS,), jnp.int32),
            pltpu.VMEM((SLOTS, 1000), jnp.float32),
            pltpu.VMEM((COLS_PAD,), jnp.float32),
            pltpu.SemaphoreType.DMA,
            pltpu.SemaphoreType.DMA,
        ],
    )
    return fn(hist_pad, sched, cid_time)


def kernel(history, current, cid_time):
    hist_pad = jnp.pad(history.astype(jnp.int32), (0, COLS_PAD - COLS))
    cur = jnp.pad(current.astype(jnp.int32), (0, NW * ROWS_PER_W - ROWS))
    sched = jnp.pad(cur.reshape(NW, ROWS_PER_W),
                    ((0, 0), (0, SLOTS - ROWS_PER_W)))
    return _run(hist_pad, sched, cid_time)


# speedup vs baseline: 1.5688x; 1.5688x over previous
import functools

import jax
import jax.numpy as jnp
from jax import lax
from jax.experimental import pallas as pl
from jax.experimental.pallas import tpu as pltpu
from jax.experimental.pallas import tpu_sc as plsc

L = 16
NC = 2
NS = 16
NW = NC * NS
ROWS = 50
COLS = 200
NCHUNK = (COLS + L - 1) // L
COLS_PAD = NCHUNK * L
ROWS_PER_W = 2
TABLE = 1000


def _sc_body(hist_hbm, cur_hbm, table_hbm, out_hbm,
             hist_v, cur_v, row_a, row_b, e_a, e_b,
             hsem, csem, asem, bsem, osem):
    cid = lax.axis_index("c")
    sid = lax.axis_index("s")
    wid = sid * NC + cid

    hcopy = pltpu.async_copy(hist_hbm, hist_v.at[pl.ds(0, COLS)], hsem)
    pltpu.async_copy(cur_hbm, cur_v.at[pl.ds(0, ROWS)], csem).wait()

    i0 = jnp.minimum(wid * ROWS_PER_W, ROWS - ROWS_PER_W)
    i1 = i0 + 1
    curpair = cur_v[pl.ds(i0, L)]
    r0 = curpair[0]
    r1 = curpair[1]
    acopy = pltpu.async_copy(table_hbm.at[r0], row_a, asem)
    bcopy = pltpu.async_copy(table_hbm.at[r1], row_b, bsem)
    hcopy.wait()

    lane = lax.broadcasted_iota(jnp.int32, (L,), 0)
    OVERLAP = NCHUNK * L - COLS
    out_copies = []
    for i_out, row_v, e_v, cp in ((i0, row_a, e_a, acopy),
                                  (i1, row_b, e_b, bcopy)):
        cp.wait()
        vals = []
        m = jnp.full((L,), -jnp.inf, jnp.float32)
        for c in range(NCHUNK):
            off = c * L if (c + 1) * L <= COLS else COLS - L
            idxc = hist_v[pl.ds(off, L)]
            v = plsc.load_gather(row_v, [idxc])
            if off != c * L:
                v = jnp.where(lane >= OVERLAP, v, -jnp.inf)
            vals.append(v)
            m = jnp.maximum(m, v)
        mmax = jnp.max(m)
        s = jnp.zeros((L,), jnp.float32)
        for c in range(NCHUNK):
            vals[c] = jnp.exp(vals[c] - mmax)
            s = s + vals[c]
        inv = jnp.full((L,), 1.0, jnp.float32) / jnp.broadcast_to(
            jnp.sum(s), (L,))
        e_v[pl.ds(COLS - L, L)] = vals[NCHUNK - 1] * inv
        for c in range(NCHUNK - 1):
            e_v[pl.ds(c * L, L)] = vals[c] * inv
        out_copies.append(
            pltpu.async_copy(e_v,
                             out_hbm.at[pl.ds(i_out * COLS_PAD, COLS_PAD)],
                             osem))

    for cp in out_copies:
        cp.wait()


@jax.jit
def _run(history, current, cid_time):
    mesh = plsc.VectorSubcoreMesh(
        core_axis_name="c", subcore_axis_name="s",
        num_cores=NC, num_subcores=NS)
    fn = pl.kernel(
        _sc_body,
        out_type=jax.ShapeDtypeStruct((ROWS * COLS_PAD,), jnp.float32),
        mesh=mesh,
        compiler_params=pltpu.CompilerParams(
            needs_layout_passes=False,
        ),
        scratch_types=[
            pltpu.VMEM((COLS_PAD,), jnp.int32),
            pltpu.VMEM((ROWS - ROWS_PER_W + L,), jnp.int32),
            pltpu.VMEM((TABLE,), jnp.float32),
            pltpu.VMEM((TABLE,), jnp.float32),
            pltpu.VMEM((COLS_PAD,), jnp.float32),
            pltpu.VMEM((COLS_PAD,), jnp.float32),
            pltpu.SemaphoreType.DMA,
            pltpu.SemaphoreType.DMA,
            pltpu.SemaphoreType.DMA,
            pltpu.SemaphoreType.DMA,
            pltpu.SemaphoreType.DMA,
        ],
    )
    out_flat = fn(history, current, cid_time)
    return out_flat.reshape(ROWS, COLS_PAD)[:, :COLS]


def kernel(history, current, cid_time):
    return _run(history.astype(jnp.int32), current.astype(jnp.int32),
                cid_time)


# speedup vs baseline: 1.5738x; 1.0032x over previous
import functools

import jax
import jax.numpy as jnp
from jax import lax
from jax.experimental import pallas as pl
from jax.experimental.pallas import tpu as pltpu
from jax.experimental.pallas import tpu_sc as plsc

L = 16
NC = 2
NS = 16
NW = NC * NS
ROWS = 50
COLS = 200
NCHUNK = (COLS + L - 1) // L
COLS_PAD = NCHUNK * L
ROWS_PER_W = 2
TABLE = 1000


def _sc_body(hist_hbm, cur_hbm, table_hbm, out_hbm,
             hist_v, cur_v, row_a, row_b, e_a, e_b,
             hsem, csem, asem, bsem, osem):
    cid = lax.axis_index("c")
    sid = lax.axis_index("s")
    wid = sid * NC + cid

    hcopy = pltpu.async_copy(hist_hbm, hist_v.at[pl.ds(0, COLS)], hsem)
    pltpu.async_copy(cur_hbm, cur_v.at[pl.ds(0, ROWS)], csem).wait()

    i0 = jnp.minimum(wid * ROWS_PER_W, ROWS - ROWS_PER_W)
    i1 = i0 + 1
    curpair = cur_v[pl.ds(i0, L)]
    r0 = curpair[0]
    r1 = curpair[1]
    acopy = pltpu.async_copy(table_hbm.at[r0], row_a, asem)
    bcopy = pltpu.async_copy(table_hbm.at[r1], row_b, bsem)
    hcopy.wait()

    lane = lax.broadcasted_iota(jnp.int32, (L,), 0)
    OVERLAP = NCHUNK * L - COLS
    out_copies = []
    for i_out, row_v, e_v, cp in ((i0, row_a, e_a, acopy),
                                  (i1, row_b, e_b, bcopy)):
        cp.wait()
        vals = []
        m = jnp.full((L,), -jnp.inf, jnp.float32)
        for c in range(NCHUNK):
            off = c * L if (c + 1) * L <= COLS else COLS - L
            idxc = hist_v[pl.ds(off, L)]
            v = plsc.load_gather(row_v, [idxc])
            if off != c * L:
                v = jnp.where(lane >= OVERLAP, v, -jnp.inf)
            vals.append(v)
            m = jnp.maximum(m, v)
        mmax = jnp.max(m)
        s = jnp.zeros((L,), jnp.float32)
        for c in range(NCHUNK):
            vals[c] = jnp.exp(vals[c] - mmax)
            s = s + vals[c]
        inv = jnp.full((L,), 1.0, jnp.float32) / jnp.broadcast_to(
            jnp.sum(s), (L,))
        e_v[pl.ds(COLS - L, L)] = vals[NCHUNK - 1] * inv
        for c in range(NCHUNK - 1):
            e_v[pl.ds(c * L, L)] = vals[c] * inv
        out_copies.append(
            pltpu.async_copy(e_v.at[pl.ds(0, COLS)],
                             out_hbm.at[pl.ds(i_out * COLS, COLS)],
                             osem))

    for cp in out_copies:
        cp.wait()


@jax.jit
def _run(history, current, cid_time):
    mesh = plsc.VectorSubcoreMesh(
        core_axis_name="c", subcore_axis_name="s",
        num_cores=NC, num_subcores=NS)
    fn = pl.kernel(
        _sc_body,
        out_type=jax.ShapeDtypeStruct((ROWS * COLS,), jnp.float32),
        mesh=mesh,
        compiler_params=pltpu.CompilerParams(
            needs_layout_passes=False,
        ),
        scratch_types=[
            pltpu.VMEM((COLS_PAD,), jnp.int32),
            pltpu.VMEM((ROWS - ROWS_PER_W + L,), jnp.int32),
            pltpu.VMEM((TABLE,), jnp.float32),
            pltpu.VMEM((TABLE,), jnp.float32),
            pltpu.VMEM((COLS_PAD,), jnp.float32),
            pltpu.VMEM((COLS_PAD,), jnp.float32),
            pltpu.SemaphoreType.DMA,
            pltpu.SemaphoreType.DMA,
            pltpu.SemaphoreType.DMA,
            pltpu.SemaphoreType.DMA,
            pltpu.SemaphoreType.DMA,
        ],
    )
    out_flat = fn(history, current, cid_time)
    return out_flat.reshape(ROWS, COLS)


def kernel(history, current, cid_time):
    return _run(history.astype(jnp.int32), current.astype(jnp.int32),
                cid_time)


# speedup vs baseline: 1.5919x; 1.0115x over previous
import functools

import jax
import jax.numpy as jnp
from jax import lax
from jax.experimental import pallas as pl
from jax.experimental.pallas import tpu as pltpu
from jax.experimental.pallas import tpu_sc as plsc

L = 16
NC = 2
NS = 16
NW = NC * NS
ROWS = 50
COLS = 200
NCHUNK = (COLS + L - 1) // L
COLS_PAD = NCHUNK * L
ROWS_PER_W = 2
TABLE = 1000


def _sc_body(hist_hbm, cur_hbm, table_hbm, out_hbm,
             hist_v, cur_v, row_a, row_b, e_a, e_b,
             hsem, csem, asem, bsem, osem):
    cid = lax.axis_index("c")
    sid = lax.axis_index("s")
    wid = sid * NC + cid

    hcopy = pltpu.async_copy(hist_hbm, hist_v.at[pl.ds(0, COLS)], hsem)
    pltpu.async_copy(cur_hbm, cur_v.at[pl.ds(0, ROWS)], csem).wait()

    i0 = jnp.minimum(wid * ROWS_PER_W, ROWS - ROWS_PER_W)
    i1 = i0 + 1
    curpair = cur_v[pl.ds(i0, L)]
    r0 = curpair[0]
    r1 = curpair[1]
    acopy = pltpu.async_copy(table_hbm.at[r0], row_a, asem)
    bcopy = pltpu.async_copy(table_hbm.at[r1], row_b, bsem)
    hcopy.wait()

    lane = lax.broadcasted_iota(jnp.int32, (L,), 0)
    OVERLAP = NCHUNK * L - COLS
    NFULL = NCHUNK - 1
    ninf = jnp.full((L,), -jnp.inf, jnp.float32)

    out_copies = []
    for i_out, row_v, e_v, cp in ((i0, row_a, e_a, acopy),
                                  (i1, row_b, e_b, bcopy)):
        cp.wait()
        e_v[pl.ds(NFULL * L, L)] = ninf
        vtail = plsc.load_gather(row_v, [hist_v[pl.ds(COLS - L, L)]])
        vtail = jnp.where(lane >= OVERLAP, vtail, -jnp.inf)
        e_v[pl.ds(COLS - L, L)] = vtail

        def p1(c, m):
            v = plsc.load_gather(row_v, [hist_v[pl.ds(c * L, L)]])
            e_v[pl.ds(c * L, L)] = v
            return jnp.maximum(m, v)
        m = lax.fori_loop(0, NFULL, p1, vtail)
        mmax = jnp.max(m)

        def p2(c, s):
            t = jnp.exp(e_v[pl.ds(c * L, L)] - mmax)
            e_v[pl.ds(c * L, L)] = t
            return s + t
        s = lax.fori_loop(0, NCHUNK, p2, jnp.zeros((L,), jnp.float32))
        inv = jnp.full((L,), 1.0, jnp.float32) / jnp.broadcast_to(
            jnp.sum(s), (L,))

        def p3(c, carry):
            e_v[pl.ds(c * L, L)] = e_v[pl.ds(c * L, L)] * inv
            return carry
        lax.fori_loop(0, NCHUNK, p3, jnp.int32(0))

        out_copies.append(
            pltpu.async_copy(e_v.at[pl.ds(0, COLS)],
                             out_hbm.at[pl.ds(i_out * COLS, COLS)],
                             osem))

    for cp in out_copies:
        cp.wait()


@jax.jit
def _run(history, current, cid_time):
    mesh = plsc.VectorSubcoreMesh(
        core_axis_name="c", subcore_axis_name="s",
        num_cores=NC, num_subcores=NS)
    fn = pl.kernel(
        _sc_body,
        out_type=jax.ShapeDtypeStruct((ROWS * COLS,), jnp.float32),
        mesh=mesh,
        compiler_params=pltpu.CompilerParams(
            needs_layout_passes=False,
        ),
        scratch_types=[
            pltpu.VMEM((COLS_PAD,), jnp.int32),
            pltpu.VMEM((ROWS - ROWS_PER_W + L,), jnp.int32),
            pltpu.VMEM((TABLE,), jnp.float32),
            pltpu.VMEM((TABLE,), jnp.float32),
            pltpu.VMEM((COLS_PAD,), jnp.float32),
            pltpu.VMEM((COLS_PAD,), jnp.float32),
            pltpu.SemaphoreType.DMA,
            pltpu.SemaphoreType.DMA,
            pltpu.SemaphoreType.DMA,
            pltpu.SemaphoreType.DMA,
            pltpu.SemaphoreType.DMA,
        ],
    )
    out_flat = fn(history, current, cid_time)
    return out_flat.reshape(ROWS, COLS)


def kernel(history, current, cid_time):
    return _run(history.astype(jnp.int32), current.astype(jnp.int32),
                cid_time)
